# Initial kernel scaffold; baseline (speedup 1.0000x reference)
import jax
import jax.numpy as jnp
from jax.experimental import pallas as pl


def kernel(inputs):
    raise NotImplementedError("write your pallas kernel here")


# speedup vs baseline: 5.4851x; 5.4851x over previous
import jax
import jax.numpy as jnp
from jax import lax
from jax.experimental import pallas as pl
from jax.experimental.pallas import tpu as pltpu
from jax.experimental.pallas import tpu_sc as plsc

NC = 2
NS = 16
NW = NC * NS

B_ROWS = 1024000
M = 32
K_FRAME = 2000
NSEQ = B_ROWS // K_FRAME
HALF_ROWS = K_FRAME // 2
CHUNK = HALF_ROWS * M
SEQ_PER_W = NSEQ // NW
CHUNKS_PER_W = 2 * SEQ_PER_W
GROUPS = HALF_ROWS // 3
G_UNROLL = 3
OUT_D = 6 * M
OUT_PER_W = SEQ_PER_W * OUT_D

R0 = 1.0 / 334.0
R1 = 1.0 / 333.0


def _chunk_means(buf):
    zero = jnp.zeros((16,), jnp.float32)

    def body(g, acc):
        out = list(acc)
        base = g * (G_UNROLL * 3 * M)
        for u in range(G_UNROLL):
            for p in range(3):
                for h in range(2):
                    off = base + u * 3 * M + p * M + h * 16
                    out[p * 2 + h] = out[p * 2 + h] + buf[pl.ds(off, 16)]
        return tuple(out)

    acc = lax.fori_loop(0, GROUPS // G_UNROLL, body, (zero,) * 6)
    a0 = acc[0] + buf[pl.ds((HALF_ROWS - 1) * M, 16)]
    a1 = acc[1] + buf[pl.ds((HALF_ROWS - 1) * M + 16, 16)]
    return (a0 * R0, a1 * R0, acc[2] * R1, acc[3] * R1,
            acc[4] * R1, acc[5] * R1)


def _sc_body(x_hbm, o_hbm, buf0, buf1, ob, sem0, sem1):
    wid = lax.axis_index("s") * NC + lax.axis_index("c")
    base = wid * (CHUNKS_PER_W * CHUNK)

    def start(c, buf, sem):
        pltpu.async_copy(x_hbm.at[pl.ds(base + c * CHUNK, CHUNK)], buf, sem)

    def wait(buf, sem):
        pltpu.make_async_copy(x_hbm.at[pl.ds(base, CHUNK)], buf, sem).wait()

    def compute_store(buf, c):
        m = _chunk_means(buf)
        obase = c * (3 * M)
        for k in range(6):
            ob[pl.ds(obase + k * 16, 16)] = m[k]

    start(0, buf0, sem0)
    start(1, buf1, sem1)

    def iter_body(i, carry):
        c0 = 2 * i
        wait(buf0, sem0)
        compute_store(buf0, c0)
        start(c0 + 2, buf0, sem0)
        wait(buf1, sem1)
        compute_store(buf1, c0 + 1)
        start(c0 + 3, buf1, sem1)
        return carry

    lax.fori_loop(0, CHUNKS_PER_W // 2 - 1, iter_body, 0)
    wait(buf0, sem0)
    compute_store(buf0, CHUNKS_PER_W - 2)
    wait(buf1, sem1)
    compute_store(buf1, CHUNKS_PER_W - 1)

    pltpu.sync_copy(ob, o_hbm.at[pl.ds(wid * OUT_PER_W, OUT_PER_W)])


_sc_call = pl.kernel(
    _sc_body,
    out_type=jax.ShapeDtypeStruct((NSEQ * OUT_D,), jnp.float32),
    mesh=plsc.VectorSubcoreMesh(core_axis_name="c", subcore_axis_name="s"),
    scratch_types=[
        pltpu.VMEM((CHUNK,), jnp.float32),
        pltpu.VMEM((CHUNK,), jnp.float32),
        pltpu.VMEM((OUT_PER_W,), jnp.float32),
        pltpu.SemaphoreType.DMA,
        pltpu.SemaphoreType.DMA,
    ],
)


@jax.jit
def _fll(x):
    out = _sc_call(x.reshape(-1))
    return out.reshape(NSEQ, OUT_D)


def kernel(inputs):
    return _fll(inputs)


# speedup vs baseline: 18.6420x; 3.3987x over previous
import jax
import jax.numpy as jnp
from jax import lax
from jax.experimental import pallas as pl
from jax.experimental.pallas import tpu as pltpu
from jax.experimental.pallas import tpu_sc as plsc

NC = 2
NS = 16
NW = NC * NS

B_ROWS = 1024000
M = 32
K_FRAME = 2000
NSEQ = B_ROWS // K_FRAME
HALF_ROWS = K_FRAME // 2
ROWS_PER_W = B_ROWS // NW
HALves_PER_W = ROWS_PER_W // HALF_ROWS
LANE_TILE = 128
FETCH_TILES = 9
FETCH_ROWS = FETCH_TILES * LANE_TILE
TJ_MAX = ROWS_PER_W // LANE_TILE - FETCH_TILES
BUF_WORDS = M * FETCH_ROWS
OUT_D = 6 * M
OUT_PER_W = (NSEQ // NW) * OUT_D
SG = 48
N_SG = 20

RECIP = (1.0 / 334.0, 1.0 / 333.0, 1.0 / 333.0)


def _sc_body(x_hbm, o_hbm, buf0, buf1, ob, sem0, sem1):
    wid = lax.axis_index("s") * NC + lax.axis_index("c")
    wrow0 = wid * ROWS_PER_W

    lane = lax.iota(jnp.int32, 16)
    masks = [[(lane + c) % 3 == m for m in range(3)] for c in range(3)]
    tail_masks = [masks[0][m] & (lane >= 8) for m in range(3)]
    first_lane = lane < 1
    zero16 = jnp.zeros((16,), jnp.float32)
    izero16 = jnp.zeros((16,), jnp.int32)

    def tj0_of(h):
        return jnp.minimum((h * HALF_ROWS) // LANE_TILE, TJ_MAX)

    def issue(h, buf, sem):
        g0 = wrow0 + tj0_of(h) * LANE_TILE
        for f in range(M):
            pltpu.async_copy(
                x_hbm.at[f, pl.ds(g0, FETCH_ROWS)],
                buf.at[pl.ds(f * FETCH_ROWS, FETCH_ROWS)],
                sem,
            )

    def wait_all(buf, sem):
        pltpu.make_async_copy(o_hbm.at[pl.ds(0, BUF_WORDS)], buf, sem).wait()

    def compute_half(buf, h):
        s0 = h * HALF_ROWS - tj0_of(h) * LANE_TILE
        obase = h * (3 * M)

        def feat(f, carry):
            base = f * FETCH_ROWS + s0

            def sg_body(g, acc):
                out = list(acc)
                o = base + g * SG
                for c in range(3):
                    v = buf[pl.ds(o + 16 * c, 16)]
                    for m in range(3):
                        out[m] = out[m] + jnp.where(masks[c][m], v, 0.0)
                return tuple(out)

            acc = lax.fori_loop(0, N_SG, sg_body, (zero16,) * 3)
            a = list(acc)
            v = buf[pl.ds(base + 960, 16)]
            for m in range(3):
                a[m] = a[m] + jnp.where(masks[0][m], v, 0.0)
            v = buf[pl.ds(base + 976, 16)]
            for m in range(3):
                a[m] = a[m] + jnp.where(masks[1][m], v, 0.0)
            v = buf[pl.ds(base + 984, 16)]
            for m in range(3):
                a[m] = a[m] + jnp.where(tail_masks[m], v, 0.0)
            for m in range(3):
                mean = jnp.sum(a[m]) * RECIP[m]
                idx = obase + m * M + f
                plsc.store_scatter(ob, [izero16 + idx], zero16 + mean,
                                   mask=first_lane)
            return carry

        lax.fori_loop(0, M, feat, 0)

    issue(0, buf0, sem0)
    issue(1, buf1, sem1)

    def iter_body(i, carry):
        h0 = 2 * i
        wait_all(buf0, sem0)
        compute_half(buf0, h0)
        issue(h0 + 2, buf0, sem0)
        wait_all(buf1, sem1)
        compute_half(buf1, h0 + 1)
        issue(h0 + 3, buf1, sem1)
        return carry

    lax.fori_loop(0, HALves_PER_W // 2 - 1, iter_body, 0)
    wait_all(buf0, sem0)
    compute_half(buf0, HALves_PER_W - 2)
    wait_all(buf1, sem1)
    compute_half(buf1, HALves_PER_W - 1)

    pltpu.sync_copy(ob, o_hbm.at[pl.ds(wid * OUT_PER_W, OUT_PER_W)])


_sc_call = pl.kernel(
    _sc_body,
    out_type=jax.ShapeDtypeStruct((NSEQ * OUT_D,), jnp.float32),
    mesh=plsc.VectorSubcoreMesh(core_axis_name="c", subcore_axis_name="s"),
    scratch_types=[
        pltpu.VMEM((BUF_WORDS,), jnp.float32),
        pltpu.VMEM((BUF_WORDS,), jnp.float32),
        pltpu.VMEM((OUT_PER_W,), jnp.float32),
        pltpu.SemaphoreType.DMA,
        pltpu.SemaphoreType.DMA,
    ],
    compiler_params=pltpu.CompilerParams(
        use_tc_tiling_on_sc=True, needs_layout_passes=False),
)


@jax.jit
def _fll(x):
    out = _sc_call(x.T)
    return out.reshape(NSEQ, OUT_D)


def kernel(inputs):
    return _fll(inputs)


# speedup vs baseline: 21.7888x; 1.1688x over previous
import jax
import jax.numpy as jnp
from jax import lax
from jax.experimental import pallas as pl
from jax.experimental.pallas import tpu as pltpu
from jax.experimental.pallas import tpu_sc as plsc

NC = 2
NS = 16
NW = NC * NS

B_ROWS = 1024000
M = 32
K_FRAME = 2000
NSEQ = B_ROWS // K_FRAME
HALF_ROWS = K_FRAME // 2
ROWS_PER_W = B_ROWS // NW
HALves_PER_W = ROWS_PER_W // HALF_ROWS
LANE_TILE = 128
FETCH_TILES = 9
FETCH_ROWS = FETCH_TILES * LANE_TILE
TJ_MAX = ROWS_PER_W // LANE_TILE - FETCH_TILES
BUF_WORDS = M * FETCH_ROWS
OUT_D = 6 * M
OUT_PER_W = (NSEQ // NW) * OUT_D
SG = 48
N_SG = 20

RECIP = (1.0 / 334.0, 1.0 / 333.0, 1.0 / 333.0)


def _sc_body(x_hbm, o_hbm, buf0, buf1, ob, sem0, sem1):
    wid = lax.axis_index("s") * NC + lax.axis_index("c")
    wrow0 = wid * ROWS_PER_W

    lane = lax.iota(jnp.int32, 16)
    masks = [[(lane + c) % 3 == m for m in range(3)] for c in range(3)]
    tail_masks = [masks[0][m] & (lane >= 8) for m in range(3)]
    first_lane = lane < 1
    zero16 = jnp.zeros((16,), jnp.float32)
    izero16 = jnp.zeros((16,), jnp.int32)

    def tj0_of(h):
        return jnp.minimum((h * HALF_ROWS) // LANE_TILE, TJ_MAX)

    def issue(h, buf, sem):
        g0 = wrow0 + tj0_of(h) * LANE_TILE
        for f in range(M):
            pltpu.async_copy(
                x_hbm.at[f, pl.ds(g0, FETCH_ROWS)],
                buf.at[pl.ds(f * FETCH_ROWS, FETCH_ROWS)],
                sem,
            )

    def wait_all(buf, sem):
        pltpu.make_async_copy(o_hbm.at[pl.ds(0, BUF_WORDS)], buf, sem).wait()

    def compute_half(buf, h):
        s0 = h * HALF_ROWS - tj0_of(h) * LANE_TILE
        obase = h * (3 * M)

        def feat(f, carry):
            base = f * FETCH_ROWS + s0

            def sg_body(g, acc):
                a0, a1, a2 = acc
                o = base + g * SG
                a0 = a0 + buf[pl.ds(o, 16)]
                a1 = a1 + buf[pl.ds(o + 16, 16)]
                a2 = a2 + buf[pl.ds(o + 32, 16)]
                return (a0, a1, a2)

            a0, a1, a2 = lax.fori_loop(0, N_SG, sg_body, (zero16,) * 3)
            a0 = a0 + buf[pl.ds(base + 960, 16)]
            a1 = a1 + buf[pl.ds(base + 976, 16)]
            a0 = a0 + jnp.where(lane >= 8, buf[pl.ds(base + 984, 16)], 0.0)
            acc = (a0, a1, a2)
            for m in range(3):
                w = (jnp.where(masks[0][m], a0, 0.0)
                     + jnp.where(masks[1][m], a1, 0.0)
                     + jnp.where(masks[2][m], a2, 0.0))
                mean = jnp.sum(w) * RECIP[m]
                idx = obase + m * M + f
                plsc.store_scatter(ob, [izero16 + idx], zero16 + mean,
                                   mask=first_lane)
            return carry

        lax.fori_loop(0, M, feat, 0)

    issue(0, buf0, sem0)
    issue(1, buf1, sem1)

    def iter_body(i, carry):
        h0 = 2 * i
        wait_all(buf0, sem0)
        compute_half(buf0, h0)
        issue(h0 + 2, buf0, sem0)
        wait_all(buf1, sem1)
        compute_half(buf1, h0 + 1)
        issue(h0 + 3, buf1, sem1)
        return carry

    lax.fori_loop(0, HALves_PER_W // 2 - 1, iter_body, 0)
    wait_all(buf0, sem0)
    compute_half(buf0, HALves_PER_W - 2)
    wait_all(buf1, sem1)
    compute_half(buf1, HALves_PER_W - 1)

    pltpu.sync_copy(ob, o_hbm.at[pl.ds(wid * OUT_PER_W, OUT_PER_W)])


_sc_call = pl.kernel(
    _sc_body,
    out_type=jax.ShapeDtypeStruct((NSEQ * OUT_D,), jnp.float32),
    mesh=plsc.VectorSubcoreMesh(core_axis_name="c", subcore_axis_name="s"),
    scratch_types=[
        pltpu.VMEM((BUF_WORDS,), jnp.float32),
        pltpu.VMEM((BUF_WORDS,), jnp.float32),
        pltpu.VMEM((OUT_PER_W,), jnp.float32),
        pltpu.SemaphoreType.DMA,
        pltpu.SemaphoreType.DMA,
    ],
    compiler_params=pltpu.CompilerParams(
        use_tc_tiling_on_sc=True, needs_layout_passes=False),
)


@jax.jit
def _fll(x):
    out = _sc_call(x.T)
    return out.reshape(NSEQ, OUT_D)


def kernel(inputs):
    return _fll(inputs)


# speedup vs baseline: 29.9462x; 1.3744x over previous
import jax
import jax.numpy as jnp
from jax import lax
from jax.experimental import pallas as pl
from jax.experimental.pallas import tpu as pltpu
from jax.experimental.pallas import tpu_sc as plsc

NC = 2
NS = 16
NW = NC * NS

B_ROWS = 1024000
M = 32
K_FRAME = 2000
NSEQ = B_ROWS // K_FRAME
HALF_ROWS = K_FRAME // 2
ROWS_PER_W = B_ROWS // NW
HALVES_PER_W = ROWS_PER_W // HALF_ROWS
LANE_TILE = 128
FETCH_TILES = 9
FETCH_ROWS = FETCH_TILES * LANE_TILE
TJ_MAX = ROWS_PER_W // LANE_TILE - FETCH_TILES
BUF_WORDS = M * FETCH_ROWS
OUT_D = 6 * M
OUT_PER_W = (NSEQ // NW) * OUT_D
SG = 48
N_SG = 20

RECIP = (1.0 / 334.0, 1.0 / 333.0, 1.0 / 333.0)


def _sc_body(x_hbm, o_hbm, buf0, buf1, ob, sem0, sem1):
    wid = lax.axis_index("s") * NC + lax.axis_index("c")
    wrow0 = wid * ROWS_PER_W

    lane = lax.iota(jnp.int32, 16)
    masks = [[(lane + c) % 3 == m for m in range(3)] for c in range(3)]
    first_lane = lane < 1
    zero16 = jnp.zeros((16,), jnp.float32)
    izero16 = jnp.zeros((16,), jnp.int32)

    def tj0_of(h):
        return jnp.minimum((h * HALF_ROWS) // LANE_TILE, TJ_MAX)

    def issue(h, buf, sem):
        g0 = wrow0 + tj0_of(h) * LANE_TILE
        for f in range(M):
            pltpu.async_copy(
                x_hbm.at[f, pl.ds(g0, FETCH_ROWS)],
                buf.at[pl.ds(f * FETCH_ROWS, FETCH_ROWS)],
                sem,
            )

    def wait_all(buf, sem):
        pltpu.make_async_copy(o_hbm.at[pl.ds(0, BUF_WORDS)], buf, sem).wait()

    def compute_half(buf, h):
        s0 = h * HALF_ROWS - tj0_of(h) * LANE_TILE
        obase = h * (3 * M)

        def feat(f, carry):
            base = f * FETCH_ROWS + s0

            def sg_body(g, acc):
                a0, a1, a2 = acc
                o = base + g
                a0 = a0 + buf[pl.ds(o, 16)]
                a1 = a1 + buf[pl.ds(o + 16, 16)]
                a2 = a2 + buf[pl.ds(o + 32, 16)]
                return (a0, a1, a2)

            a0, a1, a2 = plsc.parallel_loop(
                0, N_SG * SG, step=SG, carry=(zero16,) * 3, unroll=4)(sg_body)
            a0 = a0 + buf[pl.ds(base + 960, 16)]
            a1 = a1 + buf[pl.ds(base + 976, 16)]
            a0 = a0 + jnp.where(lane >= 8, buf[pl.ds(base + 984, 16)], 0.0)
            for m in range(3):
                w = (jnp.where(masks[0][m], a0, 0.0)
                     + jnp.where(masks[1][m], a1, 0.0)
                     + jnp.where(masks[2][m], a2, 0.0))
                mean = jnp.sum(w) * RECIP[m]
                idx = obase + m * M + f
                plsc.store_scatter(ob, [izero16 + idx], zero16 + mean,
                                   mask=first_lane)
            return carry

        lax.fori_loop(0, M, feat, 0)

    issue(0, buf0, sem0)
    issue(1, buf1, sem1)

    def iter_body(i, carry):
        h0 = 2 * i
        wait_all(buf0, sem0)
        compute_half(buf0, h0)
        issue(h0 + 2, buf0, sem0)
        wait_all(buf1, sem1)
        compute_half(buf1, h0 + 1)
        issue(h0 + 3, buf1, sem1)
        return carry

    lax.fori_loop(0, HALVES_PER_W // 2 - 1, iter_body, 0)
    wait_all(buf0, sem0)
    compute_half(buf0, HALVES_PER_W - 2)
    wait_all(buf1, sem1)
    compute_half(buf1, HALVES_PER_W - 1)

    pltpu.sync_copy(ob, o_hbm.at[pl.ds(wid * OUT_PER_W, OUT_PER_W)])


_sc_call = pl.kernel(
    _sc_body,
    out_type=jax.ShapeDtypeStruct((NSEQ * OUT_D,), jnp.float32),
    mesh=plsc.VectorSubcoreMesh(core_axis_name="c", subcore_axis_name="s"),
    scratch_types=[
        pltpu.VMEM((BUF_WORDS,), jnp.float32),
        pltpu.VMEM((BUF_WORDS,), jnp.float32),
        pltpu.VMEM((OUT_PER_W,), jnp.float32),
        pltpu.SemaphoreType.DMA,
        pltpu.SemaphoreType.DMA,
    ],
    compiler_params=pltpu.CompilerParams(
        use_tc_tiling_on_sc=True, needs_layout_passes=False),
)


@jax.jit
def _fll(x):
    out = _sc_call(x.T)
    return out.reshape(NSEQ, OUT_D)


def kernel(inputs):
    return _fll(inputs)
